# Initial kernel scaffold; baseline (speedup 1.0000x reference)
import jax
import jax.numpy as jnp
from jax.experimental import pallas as pl


def kernel(x, edge_index, edge_weight, W1, b1, W2, b2):
    raise NotImplementedError("write your pallas kernel here")


# speedup vs baseline: 12.6323x; 12.6323x over previous
import functools

import jax
import jax.numpy as jnp
from jax import lax
from jax.experimental import pallas as pl
from jax.experimental.pallas import tpu as pltpu
from jax.experimental.pallas import tpu_sc as plsc

_N = 10000
_E = 320000
_D = 128
_H = 16
_C = 8

_NC = 2
_NS = 16
_NW = _NC * _NS
_EPW = _E // _NW
_CHUNK = 80
_NCHUNK = _EPW // _CHUNK
_RPS = _N // _NS
_NP = 10240
_RPS1 = _NP // _NS


def _sc_mesh():
    return plsc.VectorSubcoreMesh(core_axis_name="c", subcore_axis_name="s")


def _deg_partials(col, ew, zeros1):
    @functools.partial(
        pl.kernel,
        out_type=jax.ShapeDtypeStruct((_NC, _NP), jnp.float32),
        mesh=_sc_mesh(),
        scratch_types=[
            pltpu.VMEM((_CHUNK,), jnp.int32),
            pltpu.VMEM((_CHUNK,), jnp.float32),
            pltpu.VMEM_SHARED((_NP,), jnp.float32),
        ],
    )
    def k(col_hbm, ew_hbm, z_hbm, out_hbm, col_v, ew_v, acc):
        cid = lax.axis_index("c")
        sid = lax.axis_index("s")
        wid = cid * _NS + sid
        r0 = sid * _RPS1
        pltpu.sync_copy(z_hbm.at[pl.ds(r0, _RPS1)], acc.at[pl.ds(r0, _RPS1)])
        plsc.subcore_barrier()
        base0 = wid * _EPW

        def body(c, carry):
            b = base0 + c * _CHUNK
            pltpu.sync_copy(col_hbm.at[pl.ds(b, _CHUNK)], col_v)
            pltpu.sync_copy(ew_hbm.at[pl.ds(b, _CHUNK)], ew_v)
            pltpu.sync_copy(ew_v, acc.at[col_v], add=True)
            return carry

        lax.fori_loop(0, _NCHUNK, body, 0)
        plsc.subcore_barrier()
        pltpu.sync_copy(acc.at[pl.ds(r0, _RPS1)], out_hbm.at[cid, pl.ds(r0, _RPS1)])

    return k(col, ew, zeros1)


def _edge_aggregate(h, row, col, ew, zeros16):
    @functools.partial(
        pl.kernel,
        out_type=jax.ShapeDtypeStruct((_NC, _NP, _H), jnp.float32),
        mesh=_sc_mesh(),
        scratch_types=[
            pltpu.VMEM((_CHUNK,), jnp.int32),
            pltpu.VMEM((_CHUNK,), jnp.int32),
            pltpu.VMEM((_CHUNK,), jnp.float32),
            pltpu.VMEM((_CHUNK, _H), jnp.float32),
            pltpu.VMEM_SHARED((_NP, _H), jnp.float32),
            pltpu.SemaphoreType.DMA,
        ],
        compiler_params=pltpu.CompilerParams(use_tc_tiling_on_sc=False),
    )
    def k(h_hbm, row_hbm, col_hbm, ew_hbm, z_hbm, out_hbm,
          row_v, col_v, ew_v, buf, acc, sem):
        cid = lax.axis_index("c")
        sid = lax.axis_index("s")
        wid = cid * _NS + sid
        r0 = sid * _RPS1
        pltpu.sync_copy(z_hbm.at[pl.ds(r0, _RPS1)], acc.at[pl.ds(r0, _RPS1)])
        plsc.subcore_barrier()
        base0 = wid * _EPW

        def body(c, carry):
            b = base0 + c * _CHUNK
            pltpu.sync_copy(row_hbm.at[pl.ds(b, _CHUNK)], row_v)
            pltpu.sync_copy(col_hbm.at[pl.ds(b, _CHUNK)], col_v)
            pltpu.sync_copy(ew_hbm.at[pl.ds(b, _CHUNK)], ew_v)
            pltpu.async_copy(h_hbm.at[row_v], buf, sem).wait()

            def mul(g, inner):
                ewg = ew_v[pl.ds(g * 16, 16)]
                for j in range(16):
                    bj = g * 16 + j
                    buf[bj] = buf[bj] * ewg[j]
                return inner

            lax.fori_loop(0, _CHUNK // 16, mul, 0)
            pltpu.sync_copy(buf, acc.at[col_v], add=True)
            return carry

        lax.fori_loop(0, _NCHUNK, body, 0)
        plsc.subcore_barrier()
        pltpu.sync_copy(acc.at[pl.ds(r0, _RPS1)],
                        out_hbm.at[cid, pl.ds(r0, _RPS1)])

    return k(h, row, col, ew, zeros16)


def _tc_prologue(dp, x, W1):
    def body(dp_ref, x_ref, w_ref, dinv_ref, h1s_ref):
        deg = (dp_ref[0] + dp_ref[1])[:_N] + 1.0
        dinv = jnp.where(deg > 0, lax.rsqrt(jnp.where(deg > 0, deg, 1.0)), 0.0)
        h1 = jnp.dot(x_ref[...], w_ref[...], preferred_element_type=jnp.float32)
        dinv_ref[...] = dinv
        h1s_ref[...] = h1 * dinv

    return pl.pallas_call(
        body,
        out_shape=[
            jax.ShapeDtypeStruct((_N, 1), jnp.float32),
            jax.ShapeDtypeStruct((_N, _H), jnp.float32),
        ],
    )(dp, x, W1)


def _tc_mid(p1, h1s, dinv, b1, W2):
    def body(p_ref, h1s_ref, dinv_ref, b1_ref, w2_ref, zs_ref):
        dinv = dinv_ref[...]
        agg = (p_ref[0] + p_ref[1])[:_N] + h1s_ref[...]
        r = jnp.maximum(dinv * agg + b1_ref[...], 0.0)
        z = jnp.dot(r, w2_ref[...], preferred_element_type=jnp.float32)
        zs = z * dinv
        zs_ref[...] = jnp.concatenate(
            [zs, jnp.zeros((_N, _H - _C), jnp.float32)], axis=1)

    return pl.pallas_call(
        body,
        out_shape=jax.ShapeDtypeStruct((_N, _H), jnp.float32),
    )(p1, h1s, dinv, b1, W2)


def _tc_epilogue(p2, zs, dinv, b2):
    def body(p_ref, zs_ref, dinv_ref, b2_ref, out_ref):
        agg = (p_ref[0] + p_ref[1])[:_N, :_C] + zs_ref[:, :_C]
        o = dinv_ref[...] * agg + b2_ref[...]
        m = jnp.max(o, axis=1, keepdims=True)
        e = jnp.exp(o - m)
        s = jnp.sum(e, axis=1, keepdims=True)
        out_ref[...] = o - m - jnp.log(s)

    return pl.pallas_call(
        body,
        out_shape=jax.ShapeDtypeStruct((_N, _C), jnp.float32),
    )(p2, zs, dinv, b2)


def kernel(x, edge_index, edge_weight, W1, b1, W2, b2):
    row = edge_index[0]
    col = edge_index[1]
    ew = edge_weight.astype(jnp.float32)
    zeros1 = jnp.zeros((_NP,), jnp.float32)
    zeros16 = jnp.zeros((_NP, _H), jnp.float32)

    dp = _deg_partials(col, ew, zeros1)
    dinv, h1s = _tc_prologue(dp.reshape(_NC, _NP, 1), x, W1)
    p1 = _edge_aggregate(h1s, row, col, ew, zeros16)
    zs = _tc_mid(p1, h1s, dinv, b1.reshape(1, _H), W2)
    p2 = _edge_aggregate(zs, row, col, ew, zeros16)
    return _tc_epilogue(p2, zs, dinv, b2.reshape(1, _C))


# speedup vs baseline: 55.5662x; 4.3987x over previous
import functools

import jax
import jax.numpy as jnp
from jax import lax
from jax.experimental import pallas as pl
from jax.experimental.pallas import tpu as pltpu
from jax.experimental.pallas import tpu_sc as plsc

_N = 10000
_E = 320000
_D = 128
_H = 16
_C = 8

_NC = 2
_NS = 16
_NW = _NC * _NS
_EPW = _E // _NW
_CHUNK = 80
_NCHUNK = _EPW // _CHUNK
_NBUF = 5
_OUTER = _NCHUNK // _NBUF
_NP = 10240
_RPS1 = _NP // _NS


def _sc_mesh():
    return plsc.VectorSubcoreMesh(core_axis_name="c", subcore_axis_name="s")


def _deg_partials(col3, ew3, zeros1):
    @functools.partial(
        pl.kernel,
        out_type=jax.ShapeDtypeStruct((_NC, _NP), jnp.float32),
        mesh=_sc_mesh(),
        scratch_types=[
            pltpu.VMEM((_NCHUNK, _CHUNK), jnp.int32),
            pltpu.VMEM((_NCHUNK, _CHUNK), jnp.float32),
            pltpu.VMEM_SHARED((_NP,), jnp.float32),
            pltpu.SemaphoreType.DMA,
        ],
    )
    def k(col_hbm, ew_hbm, z_hbm, out_hbm, col_v, ew_v, acc, ssem):
        cid = lax.axis_index("c")
        sid = lax.axis_index("s")
        wid = cid * _NS + sid
        r0 = sid * _RPS1
        pltpu.sync_copy(z_hbm.at[pl.ds(r0, _RPS1)], acc.at[pl.ds(r0, _RPS1)])
        pltpu.sync_copy(col_hbm.at[wid], col_v)
        pltpu.sync_copy(ew_hbm.at[wid], ew_v)
        plsc.subcore_barrier()

        def body(c, carry):
            pltpu.async_copy(ew_v.at[c], acc.at[col_v.at[c]], ssem, add=True)
            return carry

        lax.fori_loop(0, _NCHUNK, body, 0)

        def drain(c, carry):
            pltpu.make_async_copy(ew_v.at[0], acc.at[col_v.at[0]], ssem).wait()
            return carry

        lax.fori_loop(0, _NCHUNK, drain, 0)
        plsc.subcore_barrier()
        pltpu.sync_copy(acc.at[pl.ds(r0, _RPS1)], out_hbm.at[cid, pl.ds(r0, _RPS1)])

    return k(col3, ew3, zeros1)


def _edge_aggregate(h, row3, col3, ew3, zeros16):
    @functools.partial(
        pl.kernel,
        out_type=jax.ShapeDtypeStruct((_NC, _NP, _H), jnp.float32),
        mesh=_sc_mesh(),
        scratch_types=[
            pltpu.VMEM((_NCHUNK, _CHUNK), jnp.int32),
            pltpu.VMEM((_NCHUNK, _CHUNK), jnp.int32),
            pltpu.VMEM((_NCHUNK, _CHUNK), jnp.float32),
            pltpu.VMEM((_NBUF, _CHUNK, _H), jnp.float32),
            pltpu.VMEM((_NBUF, _CHUNK, _H), jnp.float32),
            pltpu.VMEM_SHARED((_NP, _H), jnp.float32),
            pltpu.SemaphoreType.DMA((_NBUF,)),
            pltpu.SemaphoreType.DMA((_NBUF,)),
        ],
        compiler_params=pltpu.CompilerParams(use_tc_tiling_on_sc=False),
    )
    def k(h_hbm, row_hbm, col_hbm, ew_hbm, z_hbm, out_hbm,
          row_v, col_v, ew_v, gbuf, sbuf, acc, gsem, ssem):
        cid = lax.axis_index("c")
        sid = lax.axis_index("s")
        wid = cid * _NS + sid
        r0 = sid * _RPS1
        pltpu.sync_copy(z_hbm.at[pl.ds(r0, _RPS1)], acc.at[pl.ds(r0, _RPS1)])
        pltpu.sync_copy(row_hbm.at[wid], row_v)
        pltpu.sync_copy(col_hbm.at[wid], col_v)
        pltpu.sync_copy(ew_hbm.at[wid], ew_v)
        plsc.subcore_barrier()

        def gather(c, b):
            return pltpu.make_async_copy(
                h_hbm.at[row_v.at[c]], gbuf.at[b], gsem.at[b])

        def scatter(c, b):
            return pltpu.make_async_copy(
                sbuf.at[b], acc.at[col_v.at[c]], ssem.at[b])

        for b in range(_NBUF):
            gather(b, b).start()

        def outer(o, carry):
            for b in range(_NBUF):
                c = o * _NBUF + b
                gather(c, b).wait()

                @pl.when(o < _OUTER - 1)
                def _():
                    gather(c + _NBUF, b).start()

                @pl.when(o > 0)
                def _():
                    scatter(c, b).wait()

                for g in range(_CHUNK // 16):
                    ewg = ew_v[c, pl.ds(g * 16, 16)]
                    for j in range(16):
                        e = g * 16 + j
                        sbuf[b, e] = gbuf[b, e] * ewg[j]
                scatter(c, b).start(add=True)
            return carry

        lax.fori_loop(0, _OUTER, outer, 0)
        for b in range(_NBUF):
            scatter(b, b).wait()
        plsc.subcore_barrier()
        pltpu.sync_copy(acc.at[pl.ds(r0, _RPS1)],
                        out_hbm.at[cid, pl.ds(r0, _RPS1)])

    return k(h, row3, col3, ew3, zeros16)


def _tc_prologue(dp, x, W1):
    def body(dp_ref, x_ref, w_ref, dinv_ref, h1s_ref):
        deg = (dp_ref[0] + dp_ref[1])[:_N] + 1.0
        dinv = jnp.where(deg > 0, lax.rsqrt(jnp.where(deg > 0, deg, 1.0)), 0.0)
        h1 = jnp.dot(x_ref[...], w_ref[...], preferred_element_type=jnp.float32)
        dinv_ref[...] = dinv
        h1s_ref[...] = h1 * dinv

    return pl.pallas_call(
        body,
        out_shape=[
            jax.ShapeDtypeStruct((_N, 1), jnp.float32),
            jax.ShapeDtypeStruct((_N, _H), jnp.float32),
        ],
    )(dp, x, W1)


def _tc_mid(p1, h1s, dinv, b1, W2):
    def body(p_ref, h1s_ref, dinv_ref, b1_ref, w2_ref, zs_ref):
        dinv = dinv_ref[...]
        agg = (p_ref[0] + p_ref[1])[:_N] + h1s_ref[...]
        r = jnp.maximum(dinv * agg + b1_ref[...], 0.0)
        z = jnp.dot(r, w2_ref[...], preferred_element_type=jnp.float32)
        zs = z * dinv
        zs_ref[...] = jnp.concatenate(
            [zs, jnp.zeros((_N, _H - _C), jnp.float32)], axis=1)

    return pl.pallas_call(
        body,
        out_shape=jax.ShapeDtypeStruct((_N, _H), jnp.float32),
    )(p1, h1s, dinv, b1, W2)


def _tc_epilogue(p2, zs, dinv, b2):
    def body(p_ref, zs_ref, dinv_ref, b2_ref, out_ref):
        agg = (p_ref[0] + p_ref[1])[:_N, :_C] + zs_ref[:, :_C]
        o = dinv_ref[...] * agg + b2_ref[...]
        m = jnp.max(o, axis=1, keepdims=True)
        e = jnp.exp(o - m)
        s = jnp.sum(e, axis=1, keepdims=True)
        out_ref[...] = o - m - jnp.log(s)

    return pl.pallas_call(
        body,
        out_shape=jax.ShapeDtypeStruct((_N, _C), jnp.float32),
    )(p2, zs, dinv, b2)


def kernel(x, edge_index, edge_weight, W1, b1, W2, b2):
    row3 = edge_index[0].reshape(_NW, _NCHUNK, _CHUNK)
    col3 = edge_index[1].reshape(_NW, _NCHUNK, _CHUNK)
    ew3 = edge_weight.astype(jnp.float32).reshape(_NW, _NCHUNK, _CHUNK)
    zeros1 = jnp.zeros((_NP,), jnp.float32)
    zeros16 = jnp.zeros((_NP, _H), jnp.float32)

    dp = _deg_partials(col3, ew3, zeros1)
    dinv, h1s = _tc_prologue(dp.reshape(_NC, _NP, 1), x, W1)
    p1 = _edge_aggregate(h1s, row3, col3, ew3, zeros16)
    zs = _tc_mid(p1, h1s, dinv, b1.reshape(1, _H), W2)
    p2 = _edge_aggregate(zs, row3, col3, ew3, zeros16)
    return _tc_epilogue(p2, zs, dinv, b2.reshape(1, _C))


# speedup vs baseline: 91.8020x; 1.6521x over previous
import functools

import jax
import jax.numpy as jnp
from jax import lax
from jax.experimental import pallas as pl
from jax.experimental.pallas import tpu as pltpu
from jax.experimental.pallas import tpu_sc as plsc

_N = 10000
_E = 320000
_D = 128
_H = 16
_C = 8

_NC = 2
_NS = 16
_NW = _NC * _NS
_CHUNK = 128
_TCHUNK = _E // _CHUNK
_MAXCW = (_TCHUNK + _NW - 1) // _NW + 1
_NBUF = 5
_OUTER = (_MAXCW + _NBUF - 1) // _NBUF
_NP = 10240
_RPS1 = _NP // _NS
_PK = _NP // 8


def _sc_mesh():
    return plsc.VectorSubcoreMesh(core_axis_name="c", subcore_axis_name="s")


def _deg_partials(ei3, ew2, zeros1):
    @functools.partial(
        pl.kernel,
        out_type=jax.ShapeDtypeStruct((_NC, _NP, _H), jnp.float32),
        mesh=_sc_mesh(),
        scratch_types=[
            pltpu.VMEM((_MAXCW, _CHUNK), jnp.int32),
            pltpu.VMEM((_MAXCW, _CHUNK), jnp.float32),
            pltpu.VMEM((_RPS1,), jnp.float32),
            pltpu.VMEM((_RPS1, _H), jnp.float32),
            pltpu.VMEM_SHARED((_NP,), jnp.float32),
            pltpu.SemaphoreType.DMA,
        ],
        compiler_params=pltpu.CompilerParams(use_tc_tiling_on_sc=False),
    )
    def k(ei_hbm, ew_hbm, z_hbm, out_hbm, col_v, ew_v, degv, expb, acc, ssem):
        cid = lax.axis_index("c")
        sid = lax.axis_index("s")
        wid = cid * _NS + sid
        lo = wid * _TCHUNK // _NW
        ncw = (wid + 1) * _TCHUNK // _NW - lo
        r0 = sid * _RPS1
        pltpu.sync_copy(z_hbm.at[pl.ds(r0, _RPS1)], acc.at[pl.ds(r0, _RPS1)])
        pltpu.sync_copy(ei_hbm.at[pl.ds(lo, _MAXCW), 1], col_v)
        pltpu.sync_copy(ew_hbm.at[pl.ds(lo, _MAXCW)], ew_v)
        plsc.subcore_barrier()

        def body(c, carry):
            pltpu.async_copy(ew_v.at[c], acc.at[col_v.at[c]], ssem, add=True)
            return carry

        lax.fori_loop(0, ncw, body, 0)

        def drain(c, carry):
            pltpu.make_async_copy(ew_v.at[0], acc.at[col_v.at[0]], ssem).wait()
            return carry

        lax.fori_loop(0, ncw, drain, 0)
        plsc.subcore_barrier()
        pltpu.sync_copy(acc.at[pl.ds(r0, _RPS1)], degv)
        for g in range(_RPS1 // 16):
            dv = degv[pl.ds(g * 16, 16)]
            for j in range(16):
                expb[g * 16 + j] = jnp.broadcast_to(dv[j], (16,))
        pltpu.sync_copy(expb, out_hbm.at[cid, pl.ds(r0, _RPS1)])

    return k(ei3, ew2, zeros1)


def _edge_aggregate(table, ei3, ew2, zeros16):
    @functools.partial(
        pl.kernel,
        out_type=jax.ShapeDtypeStruct((_NC, _NP, _H), jnp.float32),
        mesh=_sc_mesh(),
        scratch_types=[
            pltpu.VMEM((_MAXCW, _CHUNK), jnp.int32),
            pltpu.VMEM((_MAXCW, _CHUNK), jnp.int32),
            pltpu.VMEM((_MAXCW, _CHUNK), jnp.float32),
            pltpu.VMEM((_NBUF, _CHUNK, _H), jnp.float32),
            pltpu.VMEM((_NBUF, _CHUNK, _H), jnp.float32),
            pltpu.VMEM_SHARED((_NP, _H), jnp.float32),
            pltpu.SemaphoreType.DMA((_NBUF,)),
            pltpu.SemaphoreType.DMA((_NBUF,)),
        ],
        compiler_params=pltpu.CompilerParams(use_tc_tiling_on_sc=False),
    )
    def k(h_hbm, ei_hbm, ew_hbm, z_hbm, out_hbm,
          row_v, col_v, ew_v, gbuf, sbuf, acc, gsem, ssem):
        cid = lax.axis_index("c")
        sid = lax.axis_index("s")
        wid = cid * _NS + sid
        lo = wid * _TCHUNK // _NW
        ncw = (wid + 1) * _TCHUNK // _NW - lo
        r0 = sid * _RPS1
        pltpu.sync_copy(z_hbm.at[pl.ds(r0, _RPS1)], acc.at[pl.ds(r0, _RPS1)])
        pltpu.sync_copy(ei_hbm.at[pl.ds(lo, _MAXCW), 0], row_v)
        pltpu.sync_copy(ei_hbm.at[pl.ds(lo, _MAXCW), 1], col_v)
        pltpu.sync_copy(ew_hbm.at[pl.ds(lo, _MAXCW)], ew_v)
        plsc.subcore_barrier()

        def gather(c, b):
            return pltpu.make_async_copy(
                h_hbm.at[row_v.at[c]], gbuf.at[b], gsem.at[b])

        def scatter(c, b):
            return pltpu.make_async_copy(
                sbuf.at[b], acc.at[col_v.at[c]], ssem.at[b])

        for b in range(_NBUF):
            gather(b, b).start()

        def outer(o, carry):
            for b in range(_NBUF):
                c = o * _NBUF + b

                @pl.when(c < ncw)
                def _():
                    gather(c, b).wait()

                @pl.when(c + _NBUF < ncw)
                def _():
                    gather(c + _NBUF, b).start()

                @pl.when(jnp.logical_and(o > 0, c < ncw))
                def _():
                    scatter(c, b).wait()

                @pl.when(c < ncw)
                def _():
                    for g in range(_CHUNK // 16):
                        ewg = ew_v[c, pl.ds(g * 16, 16)]
                        for j in range(16):
                            e = g * 16 + j
                            sbuf[b, e] = gbuf[b, e] * ewg[j]
                    scatter(c, b).start(add=True)
            return carry

        lax.fori_loop(0, _OUTER, outer, 0)
        for b in range(_NBUF):
            scatter(b, b).wait()
        plsc.subcore_barrier()
        pltpu.sync_copy(acc.at[pl.ds(r0, _RPS1)],
                        out_hbm.at[cid, pl.ds(r0, _RPS1)])

    return k(table, ei3, ew2, zeros16)


def _tc_matmul(xp, W1bd):
    def body(x_ref, w_ref, h_ref):
        hp = jnp.dot(x_ref[...], w_ref[...], preferred_element_type=jnp.float32)
        h_ref[...] = jnp.concatenate(
            [hp, jnp.zeros((_PK - _N // 8, 128), jnp.float32)], axis=0)

    return pl.pallas_call(
        body,
        out_shape=jax.ShapeDtypeStruct((_PK, 128), jnp.float32),
    )(xp, W1bd)


def _tc_scale(dp16, h1p):
    def body(dp_ref, h_ref, dinv_ref, t1_ref, s1_ref):
        deg = dp_ref[0] + dp_ref[1] + 1.0
        dinv = jnp.where(deg > 0, lax.rsqrt(jnp.where(deg > 0, deg, 1.0)), 0.0)
        h = h_ref[...]
        dinv_ref[...] = dinv
        t1_ref[...] = h * dinv
        s1_ref[...] = h * dinv * dinv

    return pl.pallas_call(
        body,
        out_shape=[
            jax.ShapeDtypeStruct((_PK, 128), jnp.float32),
            jax.ShapeDtypeStruct((_PK, 128), jnp.float32),
            jax.ShapeDtypeStruct((_PK, 128), jnp.float32),
        ],
    )(dp16, h1p)


def _tc_mid(p1p, dinvp, s1p, b1t, Wmid):
    def body(p_ref, d_ref, s_ref, b_ref, w_ref, out_ref):
        d = d_ref[...]
        v = jnp.maximum(d * (p_ref[0] + p_ref[1]) + s_ref[...] + b_ref[...],
                        0.0)
        out_ref[...] = d * jnp.dot(v, w_ref[...],
                                   preferred_element_type=jnp.float32)

    return pl.pallas_call(
        body,
        out_shape=jax.ShapeDtypeStruct((_PK, 128), jnp.float32),
    )(p1p, dinvp, s1p, b1t, Wmid)


def _tc_combine(p2p, dinvp, t2p, b2t):
    def body(p_ref, d_ref, t_ref, b_ref, out_ref):
        out_ref[...] = (d_ref[...] * (p_ref[0] + p_ref[1] + t_ref[...])
                        + b_ref[...])

    return pl.pallas_call(
        body,
        out_shape=jax.ShapeDtypeStruct((_PK, 128), jnp.float32),
    )(p2p, dinvp, t2p, b2t)


def _tc_softmax(o):
    def body(o_ref, out_ref):
        o = o_ref[:_N, :_C]
        m = jnp.max(o, axis=1, keepdims=True)
        e = jnp.exp(o - m)
        s = jnp.sum(e, axis=1, keepdims=True)
        out_ref[...] = o - m - jnp.log(s)

    return pl.pallas_call(
        body,
        out_shape=jax.ShapeDtypeStruct((_N, _C), jnp.float32),
    )(o)


def kernel(x, edge_index, edge_weight, W1, b1, W2, b2):
    f32 = jnp.float32
    ei3 = edge_index.reshape(2, _TCHUNK, _CHUNK).transpose(1, 0, 2)
    ew2 = edge_weight.astype(f32).reshape(_TCHUNK, _CHUNK)
    zeros1 = jnp.zeros((_NP,), f32)
    zeros16 = jnp.zeros((_NP, _H), f32)
    eye8 = jnp.eye(8, dtype=f32)
    W1bd = jnp.kron(eye8, W1)
    Wmid = jnp.kron(eye8, jnp.pad(W2, ((0, 0), (0, _H - _C))))
    b1t = jnp.tile(b1, 8).reshape(1, 128)
    b2t = jnp.tile(jnp.pad(b2, (0, _H - _C)), 8).reshape(1, 128)

    h1p = _tc_matmul(x.reshape(_N // 8, 8 * _D), W1bd)
    dp = _deg_partials(ei3, ew2, zeros1)
    dinvp, t1p, s1p = _tc_scale(dp.reshape(_NC, _PK, 128), h1p)
    p1 = _edge_aggregate(t1p.reshape(_NP, _H), ei3, ew2, zeros16)
    t2p = _tc_mid(p1.reshape(_NC, _PK, 128), dinvp, s1p, b1t, Wmid)
    p2 = _edge_aggregate(t2p.reshape(_NP, _H), ei3, ew2, zeros16)
    op = _tc_combine(p2.reshape(_NC, _PK, 128), dinvp, t2p, b2t)
    return _tc_softmax(op.reshape(_NP, _H))


# speedup vs baseline: 98.3445x; 1.0713x over previous
import functools

import jax
import jax.numpy as jnp
from jax import lax
from jax.experimental import pallas as pl
from jax.experimental.pallas import tpu as pltpu
from jax.experimental.pallas import tpu_sc as plsc

_N = 10000
_E = 320000
_D = 128
_H = 16
_C = 8

_NC = 2
_NS = 16
_NW = _NC * _NS
_CHUNK = 128
_TCHUNK = _E // _CHUNK
_MAXCW = (_TCHUNK + _NW - 1) // _NW + 1
_NBUF = 5
_OUTER = (_MAXCW + _NBUF - 1) // _NBUF
_NP = 10240
_RPS1 = _NP // _NS
_PK = _NP // 8


def _sc_mesh():
    return plsc.VectorSubcoreMesh(core_axis_name="c", subcore_axis_name="s")


def _deg_partials(ei3, ew2, zeros1):
    @functools.partial(
        pl.kernel,
        out_type=jax.ShapeDtypeStruct((_NC, _NP, _H), jnp.float32),
        mesh=_sc_mesh(),
        scratch_types=[
            pltpu.VMEM((_MAXCW, _CHUNK), jnp.int32),
            pltpu.VMEM((_MAXCW, _CHUNK), jnp.float32),
            pltpu.VMEM((_RPS1,), jnp.float32),
            pltpu.VMEM((_RPS1, _H), jnp.float32),
            pltpu.VMEM_SHARED((_NP,), jnp.float32),
            pltpu.SemaphoreType.DMA,
        ],
        compiler_params=pltpu.CompilerParams(use_tc_tiling_on_sc=False),
    )
    def k(ei_hbm, ew_hbm, z_hbm, out_hbm, col_v, ew_v, degv, expb, acc, ssem):
        cid = lax.axis_index("c")
        sid = lax.axis_index("s")
        wid = cid * _NS + sid
        lo = wid * _TCHUNK // _NW
        ncw = (wid + 1) * _TCHUNK // _NW - lo
        r0 = sid * _RPS1
        pltpu.sync_copy(z_hbm.at[pl.ds(r0, _RPS1)], acc.at[pl.ds(r0, _RPS1)])
        pltpu.sync_copy(ei_hbm.at[pl.ds(lo, _MAXCW), 1], col_v)
        pltpu.sync_copy(ew_hbm.at[pl.ds(lo, _MAXCW)], ew_v)
        plsc.subcore_barrier()

        def body(c, carry):
            pltpu.async_copy(ew_v.at[c], acc.at[col_v.at[c]], ssem, add=True)
            return carry

        lax.fori_loop(0, ncw, body, 0)

        def drain(c, carry):
            pltpu.make_async_copy(ew_v.at[0], acc.at[col_v.at[0]], ssem).wait()
            return carry

        lax.fori_loop(0, ncw, drain, 0)
        plsc.subcore_barrier()
        pltpu.sync_copy(acc.at[pl.ds(r0, _RPS1)], degv)
        for g in range(_RPS1 // 16):
            dv = degv[pl.ds(g * 16, 16)]
            for j in range(16):
                expb[g * 16 + j] = jnp.broadcast_to(dv[j], (16,))
        pltpu.sync_copy(expb, out_hbm.at[cid, pl.ds(r0, _RPS1)])

    return k(ei3, ew2, zeros1)


def _edge_aggregate(table, ei3, ew2, zeros16):
    @functools.partial(
        pl.kernel,
        out_type=jax.ShapeDtypeStruct((_NC, _NP, _H), jnp.float32),
        mesh=_sc_mesh(),
        scratch_types=[
            pltpu.VMEM((_MAXCW, _CHUNK), jnp.int32),
            pltpu.VMEM((_MAXCW, _CHUNK), jnp.int32),
            pltpu.VMEM((_MAXCW, _CHUNK), jnp.float32),
            pltpu.VMEM((_NBUF, _CHUNK, _H), jnp.float32),
            pltpu.VMEM((_NBUF, _CHUNK, _H), jnp.float32),
            pltpu.VMEM_SHARED((_NP, _H), jnp.float32),
            pltpu.SemaphoreType.DMA((_NBUF,)),
            pltpu.SemaphoreType.DMA((_NBUF,)),
        ],
        compiler_params=pltpu.CompilerParams(use_tc_tiling_on_sc=False),
    )
    def k(h_hbm, ei_hbm, ew_hbm, z_hbm, out_hbm,
          row_v, col_v, ew_v, gbuf, sbuf, acc, gsem, ssem):
        cid = lax.axis_index("c")
        sid = lax.axis_index("s")
        wid = cid * _NS + sid
        lo = wid * _TCHUNK // _NW
        ncw = (wid + 1) * _TCHUNK // _NW - lo
        r0 = sid * _RPS1
        pltpu.sync_copy(z_hbm.at[pl.ds(r0, _RPS1)], acc.at[pl.ds(r0, _RPS1)])
        pltpu.sync_copy(ei_hbm.at[pl.ds(lo, _MAXCW), 0], row_v)
        pltpu.sync_copy(ei_hbm.at[pl.ds(lo, _MAXCW), 1], col_v)
        pltpu.sync_copy(ew_hbm.at[pl.ds(lo, _MAXCW)], ew_v)
        plsc.subcore_barrier()

        def gather(c, b):
            return pltpu.make_async_copy(
                h_hbm.at[row_v.at[c]], gbuf.at[b], gsem.at[b])

        def scatter(c, b):
            return pltpu.make_async_copy(
                sbuf.at[b], acc.at[col_v.at[c]], ssem.at[b])

        for b in range(_NBUF):
            gather(b, b).start()

        def outer(o, carry):
            for b in range(_NBUF):
                c = o * _NBUF + b

                @pl.when(c < ncw)
                def _():
                    gather(c, b).wait()

                @pl.when(c + _NBUF < ncw)
                def _():
                    gather(c + _NBUF, b).start()

                @pl.when(jnp.logical_and(o > 0, c < ncw))
                def _():
                    scatter(c, b).wait()

                @pl.when(c < ncw)
                def _():
                    for g in range(_CHUNK // 16):
                        ewg = ew_v[c, pl.ds(g * 16, 16)]
                        for j in range(16):
                            e = g * 16 + j
                            sbuf[b, e] = gbuf[b, e] * ewg[j]
                    scatter(c, b).start(add=True)
            return carry

        lax.fori_loop(0, _OUTER, outer, 0)
        for b in range(_NBUF):
            scatter(b, b).wait()
        plsc.subcore_barrier()
        pltpu.sync_copy(acc.at[pl.ds(r0, _RPS1)],
                        out_hbm.at[cid, pl.ds(r0, _RPS1)])

    return k(table, ei3, ew2, zeros16)


def _tc_matmul(xp, W1bd):
    def body(x_ref, w_ref, h_ref):
        hp = jnp.dot(x_ref[...], w_ref[...], preferred_element_type=jnp.float32)
        h_ref[...] = jnp.concatenate(
            [hp, jnp.zeros((_PK - _N // 8, 128), jnp.float32)], axis=0)

    return pl.pallas_call(
        body,
        out_shape=jax.ShapeDtypeStruct((_PK, 128), jnp.float32),
    )(xp, W1bd)


def _tc_scale(dp16, h1p):
    def body(dp_ref, h_ref, dinv_ref, t1_ref, s1_ref):
        deg = dp_ref[0] + dp_ref[1] + 1.0
        dinv = jnp.where(deg > 0, lax.rsqrt(jnp.where(deg > 0, deg, 1.0)), 0.0)
        h = h_ref[...]
        dinv_ref[...] = dinv
        t1_ref[...] = h * dinv
        s1_ref[...] = h * dinv * dinv

    return pl.pallas_call(
        body,
        out_shape=[
            jax.ShapeDtypeStruct((_PK, 128), jnp.float32),
            jax.ShapeDtypeStruct((_PK, 128), jnp.float32),
            jax.ShapeDtypeStruct((_PK, 128), jnp.float32),
        ],
    )(dp16, h1p)


def _tc_mid(p1p, dinvp, s1p, b1t, Wmid):
    def body(p_ref, d_ref, s_ref, b_ref, w_ref, out_ref):
        d = d_ref[...]
        v = jnp.maximum(d * (p_ref[0] + p_ref[1]) + s_ref[...] + b_ref[...],
                        0.0)
        out_ref[...] = d * jnp.dot(v, w_ref[...],
                                   preferred_element_type=jnp.float32)

    return pl.pallas_call(
        body,
        out_shape=jax.ShapeDtypeStruct((_PK, 128), jnp.float32),
    )(p1p, dinvp, s1p, b1t, Wmid)


def _tc_combine(p2p, dinvp, t2p, b2t):
    def body(p_ref, d_ref, t_ref, b_ref, out_ref):
        out_ref[...] = (d_ref[...] * (p_ref[0] + p_ref[1] + t_ref[...])
                        + b_ref[...])

    return pl.pallas_call(
        body,
        out_shape=jax.ShapeDtypeStruct((_PK, 128), jnp.float32),
    )(p2p, dinvp, t2p, b2t)


def _tc_softmax(o):
    def body(o_ref, out_ref):
        ot = o_ref[:_N, :_C].T
        m = jnp.max(ot, axis=0, keepdims=True)
        e = jnp.exp(ot - m)
        s = jnp.sum(e, axis=0, keepdims=True)
        out_ref[...] = ot - m - jnp.log(s)

    return pl.pallas_call(
        body,
        out_shape=jax.ShapeDtypeStruct((_C, _N), jnp.float32),
    )(o).T


def kernel(x, edge_index, edge_weight, W1, b1, W2, b2):
    f32 = jnp.float32
    ei3 = edge_index.reshape(2, _TCHUNK, _CHUNK).transpose(1, 0, 2)
    ew2 = edge_weight.astype(f32).reshape(_TCHUNK, _CHUNK)
    zeros1 = jnp.zeros((_NP,), f32)
    zeros16 = jnp.zeros((_NP, _H), f32)
    eye8 = jnp.eye(8, dtype=f32)
    W1bd = jnp.kron(eye8, W1)
    Wmid = jnp.kron(eye8, jnp.pad(W2, ((0, 0), (0, _H - _C))))
    b1t = jnp.tile(b1, 8).reshape(1, 128)
    b2t = jnp.tile(jnp.pad(b2, (0, _H - _C)), 8).reshape(1, 128)

    h1p = _tc_matmul(x.reshape(_N // 8, 8 * _D), W1bd)
    dp = _deg_partials(ei3, ew2, zeros1)
    dinvp, t1p, s1p = _tc_scale(dp.reshape(_NC, _PK, 128), h1p)
    p1 = _edge_aggregate(t1p.reshape(_NP, _H), ei3, ew2, zeros16)
    t2p = _tc_mid(p1.reshape(_NC, _PK, 128), dinvp, s1p, b1t, Wmid)
    p2 = _edge_aggregate(t2p.reshape(_NP, _H), ei3, ew2, zeros16)
    op = _tc_combine(p2.reshape(_NC, _PK, 128), dinvp, t2p, b2t)
    return _tc_softmax(op.reshape(_NP, _H))
